# Initial kernel scaffold; baseline (speedup 1.0000x reference)
import functools

import jax
import jax.numpy as jnp
from jax import lax
from jax.experimental import pallas as pl
from jax.experimental.pallas import tpu as pltpu
from jax.experimental.pallas import tpu_sc as plsc

MAX_CTR_F = 100000.0
SIZE_P = 64
BATCH = 4096
MAX_CLICKED = 200

_TOTAL = BATCH * MAX_CLICKED
_NC, _NS, _LANES = 2, 16, 16
_NW = _NC * _NS
_PER_W = _TOTAL // _NW
_CHUNK = 512
_NCHUNK = _PER_W // _CHUNK


def _body(ctr_hbm, table_hbm, out_hbm, cbuf, idxbuf, rbuf0, rbuf1, sem0, sem1):
    wid = lax.axis_index("s") * _NC + lax.axis_index("c")
    base = wid * _PER_W

    pltpu.sync_copy(ctr_hbm.at[pl.ds(base, _PER_W)], cbuf)

    def quant(i, carry):
        sl = pl.ds(i * _LANES, _LANES)
        idxbuf[sl] = (cbuf[sl] * MAX_CTR_F).astype(jnp.int32)
        return carry

    lax.fori_loop(0, _PER_W // _LANES, quant, 0, unroll=8)

    rbufs = (rbuf0, rbuf1)
    sems = (sem0, sem1)

    def gather(ch, buf, sem):
        src = table_hbm.at[idxbuf.at[pl.ds(ch * _CHUNK, _CHUNK)]]
        return pltpu.async_copy(src, buf, sem)

    gather(0, rbuf0, sem0)
    gather(1, rbuf1, sem1)

    def step(c, carry):
        for b in range(2):
            ch = 2 * c + b
            pltpu.make_async_copy(rbufs[b], rbufs[b], sems[b]).wait()
            pltpu.sync_copy(rbufs[b], out_hbm.at[pl.ds(base + ch * _CHUNK, _CHUNK)])

            @pl.when(ch + 2 < _NCHUNK)
            def _():
                gather(ch + 2, rbufs[b], sems[b])

        return carry

    lax.fori_loop(0, _NCHUNK // 2, step, 0)


@jax.jit
def kernel(ctr, embedding_table):
    mesh = plsc.VectorSubcoreMesh(core_axis_name="c", subcore_axis_name="s")
    k = pl.kernel(
        _body,
        jax.ShapeDtypeStruct((_TOTAL, SIZE_P), jnp.float32),
        mesh=mesh,
        scratch_types=[
            pltpu.VMEM((_PER_W,), jnp.float32),
            pltpu.VMEM((_PER_W,), jnp.int32),
            pltpu.VMEM((_CHUNK, SIZE_P), jnp.float32),
            pltpu.VMEM((_CHUNK, SIZE_P), jnp.float32),
            pltpu.SemaphoreType.DMA,
            pltpu.SemaphoreType.DMA,
        ],
    )
    out = k(ctr.reshape(-1), embedding_table)
    return out.reshape(BATCH, MAX_CLICKED, SIZE_P)


# speedup vs baseline: 4.2017x; 4.2017x over previous
import functools

import jax
import jax.numpy as jnp
from jax import lax
from jax.experimental import pallas as pl
from jax.experimental.pallas import tpu as pltpu
from jax.experimental.pallas import tpu_sc as plsc

MAX_CTR_F = 100000.0
SIZE_P = 64
BATCH = 4096
MAX_CLICKED = 200

_TOTAL = BATCH * MAX_CLICKED
_NC, _NS, _LANES = 2, 16, 16
_NW = _NC * _NS
_PER_W = _TOTAL // _NW
_CHUNK = 512
_NCHUNK = _PER_W // _CHUNK


def _body(ctr_hbm, table_hbm, out_hbm, cbuf, idxbuf, rbuf0, rbuf1, sem0, sem1):
    wid = lax.axis_index("s") * _NC + lax.axis_index("c")
    base = wid * _PER_W

    pltpu.sync_copy(ctr_hbm.at[pl.ds(base, _PER_W)], cbuf)

    def quant(i, carry):
        sl = pl.ds(i * _LANES, _LANES)
        idxbuf[sl] = (cbuf[sl] * MAX_CTR_F).astype(jnp.int32)
        return carry

    lax.fori_loop(0, _PER_W // _LANES, quant, 0, unroll=8)

    rbufs = (rbuf0, rbuf1)
    sems = (sem0, sem1)

    def gather(ch, buf, sem):
        src = table_hbm.at[idxbuf.at[pl.ds(ch * _CHUNK, _CHUNK)]]
        return pltpu.async_copy(src, buf, sem)

    gather(0, rbuf0, sem0)
    gather(1, rbuf1, sem1)

    def step(c, carry):
        for b in range(2):
            ch = 2 * c + b
            pltpu.make_async_copy(table_hbm.at[pl.ds(0, _CHUNK)], rbufs[b], sems[b]).wait()
            pltpu.sync_copy(rbufs[b], out_hbm.at[pl.ds(base + ch * _CHUNK, _CHUNK)])

            @pl.when(ch + 2 < _NCHUNK)
            def _():
                gather(ch + 2, rbufs[b], sems[b])

        return carry

    lax.fori_loop(0, _NCHUNK // 2, step, 0)


@jax.jit
def kernel(ctr, embedding_table):
    mesh = plsc.VectorSubcoreMesh(core_axis_name="c", subcore_axis_name="s")
    k = pl.kernel(
        _body,
        jax.ShapeDtypeStruct((_TOTAL, SIZE_P), jnp.float32),
        mesh=mesh,
        scratch_types=[
            pltpu.VMEM((_PER_W,), jnp.float32),
            pltpu.VMEM((_PER_W,), jnp.int32),
            pltpu.VMEM((_CHUNK, SIZE_P), jnp.float32),
            pltpu.VMEM((_CHUNK, SIZE_P), jnp.float32),
            pltpu.SemaphoreType.DMA,
            pltpu.SemaphoreType.DMA,
        ],
        compiler_params=pltpu.CompilerParams(use_tc_tiling_on_sc=False),
    )
    out = k(ctr.reshape(-1), embedding_table)
    return out.reshape(BATCH, MAX_CLICKED, SIZE_P)


# speedup vs baseline: 4.2173x; 1.0037x over previous
import functools

import jax
import jax.numpy as jnp
from jax import lax
from jax.experimental import pallas as pl
from jax.experimental.pallas import tpu as pltpu
from jax.experimental.pallas import tpu_sc as plsc
from jax.experimental import layout as jax_layout

MAX_CTR_F = 100000.0
SIZE_P = 64
BATCH = 4096
MAX_CLICKED = 200

_TOTAL = BATCH * MAX_CLICKED
_NC, _NS, _LANES = 2, 16, 16
_NW = _NC * _NS
_PER_W = _TOTAL // _NW
_CHUNK = 512
_NCHUNK = _PER_W // _CHUNK


def _body(ctr_hbm, table_hbm, out_hbm, cbuf, idxbuf, rbuf0, rbuf1, sem0, sem1):
    wid = lax.axis_index("s") * _NC + lax.axis_index("c")
    base = wid * _PER_W

    pltpu.sync_copy(ctr_hbm.at[pl.ds(base, _PER_W)], cbuf)

    def quant(i, carry):
        sl = pl.ds(i * _LANES, _LANES)
        idxbuf[sl] = (cbuf[sl] * MAX_CTR_F).astype(jnp.int32)
        return carry

    lax.fori_loop(0, _PER_W // _LANES, quant, 0, unroll=8)

    rbufs = (rbuf0, rbuf1)
    sems = (sem0, sem1)

    def gather(ch, buf, sem):
        src = table_hbm.at[idxbuf.at[pl.ds(ch * _CHUNK, _CHUNK)]]
        return pltpu.async_copy(src, buf, sem)

    gather(0, rbuf0, sem0)
    gather(1, rbuf1, sem1)

    def step(c, carry):
        for b in range(2):
            ch = 2 * c + b
            pltpu.make_async_copy(table_hbm.at[pl.ds(0, _CHUNK)], rbufs[b], sems[b]).wait()
            pltpu.sync_copy(rbufs[b], out_hbm.at[pl.ds(base + ch * _CHUNK, _CHUNK)])

            @pl.when(ch + 2 < _NCHUNK)
            def _():
                gather(ch + 2, rbufs[b], sems[b])

        return carry

    lax.fori_loop(0, _NCHUNK // 2, step, 0)


def _impl(ctr, embedding_table):
    mesh = plsc.VectorSubcoreMesh(core_axis_name="c", subcore_axis_name="s")
    k = pl.kernel(
        _body,
        jax.ShapeDtypeStruct((_TOTAL, SIZE_P), jnp.float32),
        mesh=mesh,
        scratch_types=[
            pltpu.VMEM((_PER_W,), jnp.float32),
            pltpu.VMEM((_PER_W,), jnp.int32),
            pltpu.VMEM((_CHUNK, SIZE_P), jnp.float32),
            pltpu.VMEM((_CHUNK, SIZE_P), jnp.float32),
            pltpu.SemaphoreType.DMA,
            pltpu.SemaphoreType.DMA,
        ],
        compiler_params=pltpu.CompilerParams(use_tc_tiling_on_sc=False),
    )
    out = k(ctr.reshape(-1), embedding_table)
    return out.reshape(BATCH, MAX_CLICKED, SIZE_P)


_jitted = jax.jit(_impl, out_shardings=None)


@functools.cache
def _jit_with_layout(sharding):
    fmt = jax_layout.Format(
        jax_layout.Layout(major_to_minor=(0, 1, 2), tiling=()), sharding
    )
    return jax.jit(_impl, out_shardings=fmt)


def kernel(ctr, embedding_table):
    try:
        jitted = _jit_with_layout(ctr.sharding)
    except (AttributeError, TypeError, ValueError):
        jitted = _jitted
    return jitted(ctr, embedding_table)


# speedup vs baseline: 8.1721x; 1.9378x over previous
import functools

import jax
import jax.numpy as jnp
from jax import lax
from jax.experimental import pallas as pl
from jax.experimental.pallas import tpu as pltpu
from jax.experimental.pallas import tpu_sc as plsc

MAX_CTR_F = 100000.0
SIZE_P = 64
BATCH = 4096
MAX_CLICKED = 200

_TOTAL = BATCH * MAX_CLICKED
_NC, _NS, _LANES = 2, 16, 16
_NW = _NC * _NS
_BBLK = BATCH // _NW
_NQ = MAX_CLICKED // 2
_ROWS128 = _TOTAL // 2


def _sc_body(ctr_t_hbm, table_hbm, out_hbm, cbuf, idxbuf,
             ebuf0, obuf0, ebuf1, obuf1, sem0, sem1):
    w = lax.axis_index("s") * _NC + lax.axis_index("c")
    bbase = w * _BBLK
    obase = w * (_NQ * _BBLK)

    pltpu.sync_copy(ctr_t_hbm.at[:, pl.ds(bbase, _BBLK)], cbuf)

    def quant(s, carry):
        for k in range(_BBLK // _LANES):
            sl = pl.ds(k * _LANES, _LANES)
            idxbuf[s, sl] = (cbuf[s, sl] * MAX_CTR_F).astype(jnp.int32)
        return carry

    lax.fori_loop(0, MAX_CLICKED, quant, 0, unroll=2)

    ebufs = (ebuf0, ebuf1)
    obufs = (obuf0, obuf1)
    sems = (sem0, sem1)

    def fire(q, b):
        pltpu.async_copy(table_hbm.at[idxbuf.at[2 * q]], ebufs[b], sems[b])
        pltpu.async_copy(table_hbm.at[idxbuf.at[2 * q + 1]], obufs[b], sems[b])

    def drain(b):
        dummy = table_hbm.at[pl.ds(0, _BBLK)]
        pltpu.make_async_copy(dummy, ebufs[b], sems[b]).wait()
        pltpu.make_async_copy(dummy, obufs[b], sems[b]).wait()

    fire(0, 0)
    fire(1, 1)

    def step(c, carry):
        for b in range(2):
            q = 2 * c + b
            drain(b)
            rows = pl.ds(obase + q * _BBLK, _BBLK)
            pltpu.sync_copy(ebufs[b], out_hbm.at[rows, pl.ds(0, SIZE_P)])
            pltpu.sync_copy(obufs[b], out_hbm.at[rows, pl.ds(SIZE_P, SIZE_P)])

            @pl.when(q + 2 < _NQ)
            def _():
                fire(q + 2, b)

        return carry

    lax.fori_loop(0, _NQ // 2, step, 0)


def _sc_gather(ctr_t, table):
    mesh = plsc.VectorSubcoreMesh(core_axis_name="c", subcore_axis_name="s")
    k = pl.kernel(
        _sc_body,
        jax.ShapeDtypeStruct((_ROWS128, 2 * SIZE_P), jnp.float32),
        mesh=mesh,
        scratch_types=[
            pltpu.VMEM((MAX_CLICKED, _BBLK), jnp.float32),
            pltpu.VMEM((MAX_CLICKED, _BBLK), jnp.int32),
            pltpu.VMEM((_BBLK, SIZE_P), jnp.float32),
            pltpu.VMEM((_BBLK, SIZE_P), jnp.float32),
            pltpu.VMEM((_BBLK, SIZE_P), jnp.float32),
            pltpu.VMEM((_BBLK, SIZE_P), jnp.float32),
            pltpu.SemaphoreType.DMA,
            pltpu.SemaphoreType.DMA,
        ],
        compiler_params=pltpu.CompilerParams(use_tc_tiling_on_sc=False),
    )
    return k(ctr_t, table)


def _tc_body(x_ref, o_ref):
    for q in range(_NQ):
        blk = x_ref[pl.ds(q * _BBLK, _BBLK), :].T
        o_ref[pl.ds(2 * q, 2), :, :] = blk.reshape(2, SIZE_P, _BBLK)


def _tc_transpose(x128):
    return pl.pallas_call(
        _tc_body,
        grid=(_NW,),
        in_specs=[pl.BlockSpec((_NQ * _BBLK, 2 * SIZE_P), lambda i: (i, 0))],
        out_specs=pl.BlockSpec((MAX_CLICKED, SIZE_P, _BBLK), lambda i: (0, 0, i)),
        out_shape=jax.ShapeDtypeStruct((MAX_CLICKED, SIZE_P, BATCH), jnp.float32),
    )(x128)


@jax.jit
def kernel(ctr, embedding_table):
    inter = _sc_gather(ctr.T, embedding_table)
    phys = _tc_transpose(inter)
    return phys.transpose(2, 0, 1)
